# baseline (speedup 1.0000x reference)
import functools

import jax
import jax.numpy as jnp
from jax import lax
from jax.experimental import pallas as pl
from jax.experimental.pallas import tpu as pltpu
from jax.experimental.pallas import tpu_sc as plsc

MU = 0.1
EPS = 1e-08

NC = 2
NS = 16
NW = NC * NS
LANES = 16

CHUNK = 8192


def _sc_gather_ratio(n, t_pad, nchunks):
    mesh = plsc.VectorSubcoreMesh(
        core_axis_name="c", subcore_axis_name="s",
        num_cores=NC, num_subcores=NS)

    @functools.partial(
        pl.kernel,
        out_type=jax.ShapeDtypeStruct((t_pad,), jnp.float32),
        mesh=mesh,
        scratch_types=[
            pltpu.VMEM((CHUNK,), jnp.int32),
            pltpu.VMEM((CHUNK,), jnp.int32),
            pltpu.VMEM((CHUNK,), jnp.int32),
            pltpu.VMEM((CHUNK,), jnp.int32),
            pltpu.VMEM((CHUNK,), jnp.int32),
            pltpu.VMEM((CHUNK,), jnp.float32),
            pltpu.VMEM((CHUNK,), jnp.float32),
            pltpu.VMEM((CHUNK,), jnp.float32),
            pltpu.VMEM((n,), jnp.int32),
            pltpu.VMEM((n // 128, 128), jnp.float32),
            pltpu.SemaphoreType.DMA,
        ],
        compiler_params=pltpu.CompilerParams(needs_layout_passes=False),
    )
    def sc_kernel(kflat, ti, tj, tl, out,
                  iv_v, jv_v, lv_v, idx_il, idx_ij, val_il, val_ij,
                  ratio_v, didx_v, diag_v, sem):
        wid = lax.axis_index("s") * NC + lax.axis_index("c")

        def diag_idx_body(g, _):
            vec = (lax.iota(jnp.int32, 16) + g * LANES) * (n + 1)
            didx_v[pl.ds(g * LANES, LANES)] = vec
            return 0
        lax.fori_loop(0, n // LANES, diag_idx_body, 0)

        def diag_fetch_body(g, _):
            pltpu.async_copy(
                kflat.at[didx_v.at[pl.ds(g * 128, 128)]], diag_v.at[g],
                sem).wait()
            return 0
        lax.fori_loop(0, n // 128, diag_fetch_body, 0)

        base0 = wid * (nchunks * CHUNK)

        def chunk_body(c, _):
            base = base0 + c * CHUNK
            pltpu.sync_copy(ti.at[pl.ds(base, CHUNK)], iv_v)
            pltpu.sync_copy(tj.at[pl.ds(base, CHUNK)], jv_v)
            pltpu.sync_copy(tl.at[pl.ds(base, CHUNK)], lv_v)

            def idx_body(g, _):
                t = g * LANES
                iv = iv_v[pl.ds(t, LANES)]
                row = iv * n
                idx_il[pl.ds(t, LANES)] = row + lv_v[pl.ds(t, LANES)]
                idx_ij[pl.ds(t, LANES)] = row + jv_v[pl.ds(t, LANES)]
                return 0
            lax.fori_loop(0, CHUNK // LANES, idx_body, 0)

            cp1 = pltpu.async_copy(kflat.at[idx_il], val_il, sem)
            cp2 = pltpu.async_copy(kflat.at[idx_ij], val_ij, sem)
            cp1.wait()
            cp2.wait()

            def comp_body(g, _):
                t = g * LANES
                iv = iv_v[pl.ds(t, LANES)]
                jv = jv_v[pl.ds(t, LANES)]
                lv = lv_v[pl.ds(t, LANES)]
                di = plsc.load_gather(diag_v, [iv >> 7, iv & 127])
                dj = plsc.load_gather(diag_v, [jv >> 7, jv & 127])
                dl = plsc.load_gather(diag_v, [lv >> 7, lv & 127])
                vil = val_il[pl.ds(t, LANES)]
                vij = val_ij[pl.ds(t, LANES)]
                d_il = (-2.0 * vil + di) + dl
                d_ij = (-2.0 * vij + di) + dj
                numer = jnp.maximum(MU + d_il, EPS)
                denom = jnp.maximum((2.0 * MU + d_ij) + d_il, EPS)
                ratio_v[pl.ds(t, LANES)] = denom / numer
                return 0
            lax.fori_loop(0, CHUNK // LANES, comp_body, 0)

            pltpu.sync_copy(ratio_v, out.at[pl.ds(base, CHUNK)])
            return 0

        lax.fori_loop(0, nchunks, chunk_body, 0)

    return sc_kernel


def _tc_log_sum(t, t_pad):
    cols = 1024
    rows = t_pad // cols
    block_rows = 128
    grid = rows // block_rows

    def body(x_ref, o_ref):
        b = pl.program_id(0)

        @pl.when(b == 0)
        def _():
            o_ref[0, 0] = 0.0

        x = x_ref[...]
        r = lax.broadcasted_iota(jnp.int32, (block_rows, cols), 0)
        c = lax.broadcasted_iota(jnp.int32, (block_rows, cols), 1)
        flat = (b * block_rows + r) * cols + c
        val = jnp.where(flat < t, jnp.log(x), 0.0)
        o_ref[0, 0] += jnp.sum(val)

    return pl.pallas_call(
        body,
        grid=(grid,),
        in_specs=[pl.BlockSpec((block_rows, cols), lambda b: (b, 0))],
        out_specs=pl.BlockSpec(memory_space=pltpu.SMEM),
        out_shape=jax.ShapeDtypeStruct((1, 1), jnp.float32),
    ), rows, cols


def kernel(k, triplets):
    n = k.shape[0]
    t = triplets.shape[0]
    per_round = NW * CHUNK
    nchunks = -(-t // per_round)
    t_pad = nchunks * per_round

    kflat = k.reshape(-1)
    trip = triplets
    if t_pad != t:
        trip = jnp.concatenate(
            [trip, jnp.zeros((t_pad - t, 3), jnp.int32)], axis=0)
    ti = trip[:, 0]
    tj = trip[:, 1]
    tl = trip[:, 2]

    ratio = _sc_gather_ratio(n, t_pad, nchunks)(kflat, ti, tj, tl)

    tc_call, rows, cols = _tc_log_sum(t, t_pad)
    total = tc_call(ratio.reshape(rows, cols))
    return total[0, 0]
